# baseline (speedup 1.0000x reference)
import functools

import jax
import jax.numpy as jnp
from jax import lax
from jax.experimental import pallas as pl
from jax.experimental.pallas import tpu as pltpu
from jax.experimental.pallas import tpu_sc as plsc

_VOCAB = 100000
_EMBED = 64
_CTX = 20
_HIDDEN = 128
_NB = 10
_BLK = _VOCAB // _NB


def _sc_gather_sum(idx, embed):
    mesh = plsc.VectorSubcoreMesh(core_axis_name="c", subcore_axis_name="s")

    @functools.partial(
        pl.kernel,
        out_type=jax.ShapeDtypeStruct((1, _EMBED), jnp.float32),
        mesh=mesh,
        scratch_types=[
            pltpu.VMEM((_CTX + 16,), jnp.int32),
            pltpu.VMEM((_CTX,), jnp.int32),
            pltpu.VMEM((_CTX, 2 * _EMBED), jnp.float32),
            pltpu.VMEM((1, _EMBED), jnp.float32),
            pltpu.SemaphoreType.DMA,
        ],
    )
    def k(idx_hbm, embed_hbm, out_hbm, idx_v, idx2_v, rows_v, acc_v, sem):
        wid = lax.axis_index("s") * 2 + lax.axis_index("c")

        @pl.when(wid == 0)
        def _():
            pltpu.sync_copy(idx_hbm, idx_v.at[pl.ds(0, _CTX)])
            idx2_v[pl.ds(0, 16)] = idx_v[pl.ds(0, 16)] >> 1
            idx2_v[pl.ds(_CTX - 16, 16)] = idx_v[pl.ds(_CTX - 16, 16)] >> 1
            pltpu.async_copy(embed_hbm.at[idx2_v], rows_v, sem).wait()
            for d in range(_EMBED // 16):
                acc = jnp.zeros((16,), jnp.float32)
                for r in range(_CTX):
                    par = idx_v[pl.ds(r, 16)][0] & 1
                    acc = acc + rows_v[r, pl.ds(par * _EMBED + d * 16, 16)]
                acc_v[0, pl.ds(d * 16, 16)] = acc
            pltpu.sync_copy(acc_v, out_hbm)

    return k(idx, embed.reshape(_VOCAB // 2, 2 * _EMBED))


def _tc_body(s_ref, w1_ref, b1_ref, w2_ref, b2_ref, out_ref):
    i = pl.program_id(0)
    h = lax.dot_general(s_ref[...], w1_ref[...], (((1,), (1,)), ((), ())),
                        preferred_element_type=jnp.float32)
    h = jnp.maximum(h + b1_ref[...], 0.0)
    logits = lax.dot_general(h, w2_ref[0], (((1,), (1,)), ((), ())),
                             preferred_element_type=jnp.float32)
    out_ref[pl.ds(i, 1), :] = logits + b2_ref[0]

    @pl.when(i == _NB - 1)
    def _():
        x = out_ref[...]
        m = jnp.max(x)
        lse = m + jnp.log(jnp.sum(jnp.exp(x - m)))
        out_ref[...] = x - lse


def _tc_dense(s, fc1_w, fc1_b, fc2_w, fc2_b):
    out = pl.pallas_call(
        _tc_body,
        grid=(_NB,),
        in_specs=[
            pl.BlockSpec((1, _EMBED), lambda i: (0, 0)),
            pl.BlockSpec((_HIDDEN, _EMBED), lambda i: (0, 0)),
            pl.BlockSpec((1, _HIDDEN), lambda i: (0, 0)),
            pl.BlockSpec((1, _BLK, _HIDDEN), lambda i: (i, 0, 0)),
            pl.BlockSpec((1, 1, _BLK), lambda i: (i, 0, 0)),
        ],
        out_specs=pl.BlockSpec((_NB, _BLK), lambda i: (0, 0)),
        out_shape=jax.ShapeDtypeStruct((_NB, _BLK), jnp.float32),
    )(
        s,
        fc1_w,
        fc1_b.reshape(1, _HIDDEN),
        fc2_w.reshape(_NB, _BLK, _HIDDEN),
        fc2_b.reshape(_NB, 1, _BLK),
    )
    return out.reshape(_VOCAB)


def kernel(inputs, embed, fc1_w, fc1_b, fc2_w, fc2_b):
    s = _sc_gather_sum(inputs.astype(jnp.int32), embed)
    return _tc_dense(s, fc1_w, fc1_b, fc2_w, fc2_b)


# speedup vs baseline: 1.5447x; 1.5447x over previous
import functools

import jax
import jax.numpy as jnp
from jax import lax
from jax.experimental import pallas as pl
from jax.experimental.pallas import tpu as pltpu
from jax.experimental.pallas import tpu_sc as plsc

_VOCAB = 100000
_EMBED = 64
_CTX = 20
_HIDDEN = 128
_NB = 10
_BLK = _VOCAB // _NB


def _sc_gather_sum(idx, embed):
    mesh = plsc.VectorSubcoreMesh(core_axis_name="c", subcore_axis_name="s")

    @functools.partial(
        pl.kernel,
        out_type=jax.ShapeDtypeStruct((1, _EMBED), jnp.float32),
        mesh=mesh,
        scratch_types=[
            pltpu.VMEM((_CTX + 16,), jnp.int32),
            pltpu.VMEM((_CTX, 8, _EMBED), jnp.float32),
            pltpu.VMEM((1, _EMBED), jnp.float32),
            pltpu.SemaphoreType.DMA,
        ],
    )
    def k(idx_hbm, embed_hbm, out_hbm, idx_v, rows_v, acc_v, sem):
        wid = lax.axis_index("s") * 2 + lax.axis_index("c")

        @pl.when(wid == 0)
        def _():
            pltpu.sync_copy(idx_hbm, idx_v.at[pl.ds(0, _CTX)])
            handles = []
            for r in range(_CTX):
                g = idx_v[pl.ds(r, 16)][0] >> 3
                handles.append(pltpu.async_copy(embed_hbm.at[g], rows_v.at[r], sem))
            for h in handles:
                h.wait()
            for d in range(_EMBED // 16):
                acc = jnp.zeros((16,), jnp.float32)
                for r in range(_CTX):
                    sub = idx_v[pl.ds(r, 16)][0] & 7
                    acc = acc + rows_v[r, sub, pl.ds(d * 16, 16)]
                acc_v[0, pl.ds(d * 16, 16)] = acc
            pltpu.sync_copy(acc_v, out_hbm)

    return k(idx, embed.reshape(_VOCAB // 8, 8, _EMBED))


def _tc_body(s_ref, w1_ref, b1_ref, w2_ref, b2_ref, out_ref):
    i = pl.program_id(0)
    h = lax.dot_general(s_ref[...], w1_ref[...], (((1,), (1,)), ((), ())),
                        preferred_element_type=jnp.float32)
    h = jnp.maximum(h + b1_ref[...], 0.0)
    logits = lax.dot_general(h, w2_ref[0], (((1,), (1,)), ((), ())),
                             preferred_element_type=jnp.float32)
    out_ref[pl.ds(i, 1), :] = logits + b2_ref[0]

    @pl.when(i == _NB - 1)
    def _():
        x = out_ref[...]
        m = jnp.max(x)
        lse = m + jnp.log(jnp.sum(jnp.exp(x - m)))
        out_ref[...] = x - lse


def _tc_dense(s, fc1_w, fc1_b, fc2_w, fc2_b):
    out = pl.pallas_call(
        _tc_body,
        grid=(_NB,),
        in_specs=[
            pl.BlockSpec((1, _EMBED), lambda i: (0, 0)),
            pl.BlockSpec((_HIDDEN, _EMBED), lambda i: (0, 0)),
            pl.BlockSpec((1, _HIDDEN), lambda i: (0, 0)),
            pl.BlockSpec((1, _BLK, _HIDDEN), lambda i: (i, 0, 0)),
            pl.BlockSpec((1, 1, _BLK), lambda i: (i, 0, 0)),
        ],
        out_specs=pl.BlockSpec((_NB, _BLK), lambda i: (0, 0)),
        out_shape=jax.ShapeDtypeStruct((_NB, _BLK), jnp.float32),
    )(
        s,
        fc1_w,
        fc1_b.reshape(1, _HIDDEN),
        fc2_w.reshape(_NB, _BLK, _HIDDEN),
        fc2_b.reshape(_NB, 1, _BLK),
    )
    return out.reshape(_VOCAB)


def kernel(inputs, embed, fc1_w, fc1_b, fc2_w, fc2_b):
    s = _sc_gather_sum(inputs.astype(jnp.int32), embed)
    return _tc_dense(s, fc1_w, fc1_b, fc2_w, fc2_b)
